# Initial kernel scaffold; baseline (speedup 1.0000x reference)
import jax
import jax.numpy as jnp
from jax.experimental import pallas as pl


def kernel(x, centers, W, b):
    raise NotImplementedError("write your pallas kernel here")


# speedup vs baseline: 3.1693x; 3.1693x over previous
import functools

import jax
import jax.numpy as jnp
from jax import lax
from jax.experimental import pallas as pl
from jax.experimental.pallas import tpu as pltpu
from jax.experimental.pallas import tpu_sc as plsc

C = 512
N_TOTAL = 64 * 2048
NC, NS, L = 2, 16, 16
NW = NC * NS
CHUNK = N_TOTAL // NW
NVEC = CHUNK // L


def _prep_body(ccol_ref, crow_ref, out_ref):
    cC = jnp.broadcast_to(ccol_ref[:, 0:1], (C, C))
    cR = jnp.broadcast_to(crow_ref[0:1, :], (C, C))
    iR = lax.broadcasted_iota(jnp.int32, (C, C), 1)
    iC = lax.broadcasted_iota(jnp.int32, (C, C), 0)
    before = (cR < cC) | ((cR == cC) & (iR < iC))
    rank = jnp.sum(before.astype(jnp.float32), axis=1, keepdims=True)
    rank_b = jnp.broadcast_to(rank, (C, C))
    iR_f = iR.astype(jnp.float32)
    k1 = (rank_b == iR_f).astype(jnp.float32)
    k2 = (rank_b == iR_f + 1.0).astype(jnp.float32)
    sorted_row = jnp.sum(k1 * cC, axis=0, keepdims=True)
    next_row = jnp.sum(k2 * cC, axis=0, keepdims=True)
    mid_row = 0.5 * (sorted_row + next_row)
    lane = lax.broadcasted_iota(jnp.int32, (1, C), 1)
    mid_row = jnp.where(lane == C - 1, jnp.float32(jnp.inf), mid_row)
    out_ref[...] = jnp.concatenate(
        [jnp.broadcast_to(sorted_row, (4, C)),
         jnp.broadcast_to(mid_row, (4, C))], axis=0)


_prep = pl.pallas_call(
    _prep_body,
    out_shape=jax.ShapeDtypeStruct((8, C), jnp.float32),
)


_QUANT_CACHE = []


def _get_quantize():
    if _QUANT_CACHE:
        return _QUANT_CACHE[0]

    @functools.partial(
        pl.kernel,
        mesh=plsc.VectorSubcoreMesh(core_axis_name="c", subcore_axis_name="s"),
        compiler_params=pltpu.CompilerParams(needs_layout_passes=False),
        out_type=jax.ShapeDtypeStruct((N_TOTAL,), jnp.float32),
        scratch_types=[
            pltpu.VMEM((CHUNK,), jnp.float32),
            pltpu.VMEM((C,), jnp.float32),
            pltpu.VMEM((C,), jnp.float32),
            pltpu.VMEM((CHUNK,), jnp.float32),
        ],
    )
    def _quantize(x_hbm, sorted_hbm, mid_hbm, out_hbm, xv, sv, mv, qv):
        wid = lax.axis_index("s") * NC + lax.axis_index("c")
        base = wid * CHUNK
        pltpu.sync_copy(x_hbm.at[pl.ds(base, CHUNK)], xv)
        pltpu.sync_copy(sorted_hbm, sv)
        pltpu.sync_copy(mid_hbm, mv)

        def body(i, carry):
            xvec = xv[pl.ds(i * L, L)]
            pos = jnp.zeros((L,), jnp.int32)
            step = C // 2
            while step >= 1:
                mvals = plsc.load_gather(mv, [pos + (step - 1)])
                pos = jnp.where(mvals < xvec, pos + step, pos)
                step //= 2
            qv[pl.ds(i * L, L)] = plsc.load_gather(sv, [pos])
            return carry

        lax.fori_loop(0, NVEC, body, 0)
        pltpu.sync_copy(qv, out_hbm.at[pl.ds(base, CHUNK)])

    _QUANT_CACHE.append(_quantize)
    return _quantize


def _mm_body(qx_ref, w_ref, b_ref, out_ref):
    acc = lax.dot_general(
        qx_ref[...], w_ref[...], (((1,), (1,)), ((), ())),
        preferred_element_type=jnp.float32)
    out_ref[...] = acc + jnp.broadcast_to(b_ref[0:1, :], acc.shape)


def kernel(x, centers, W, b):
    ccol = jnp.broadcast_to(centers, (C, 128))
    crow = jnp.broadcast_to(centers.reshape(1, C), (8, C))
    prep = _prep(ccol, crow)
    sorted_c = prep[0]
    mid = prep[4]
    qx = _get_quantize()(x.reshape(-1), sorted_c, mid).reshape(x.shape)
    b2 = jnp.broadcast_to(b.reshape(1, -1), (8, b.shape[0]))
    out = pl.pallas_call(
        _mm_body,
        out_shape=jax.ShapeDtypeStruct((x.shape[0], W.shape[0]), jnp.float32),
    )(qx, W, b2)
    return out


# speedup vs baseline: 4.4827x; 1.4144x over previous
import functools

import jax
import jax.numpy as jnp
from jax import lax
from jax.experimental import pallas as pl
from jax.experimental.pallas import tpu as pltpu
from jax.experimental.pallas import tpu_sc as plsc

C = 512
N_TOTAL = 64 * 2048
NC, NS, L = 2, 16, 16
NW = NC * NS
CHUNK = N_TOTAL // NW
NVEC = CHUNK // L


def _prep_body(ccol_ref, crow_ref, out_ref):
    cC = jnp.broadcast_to(ccol_ref[:, 0:1], (C, C))
    cR = jnp.broadcast_to(crow_ref[0:1, :], (C, C))
    iR = lax.broadcasted_iota(jnp.int32, (C, C), 1)
    iC = lax.broadcasted_iota(jnp.int32, (C, C), 0)
    before = (cR < cC) | ((cR == cC) & (iR < iC))
    rank = jnp.sum(before.astype(jnp.float32), axis=1, keepdims=True)
    rank_b = jnp.broadcast_to(rank, (C, C))
    iR_f = iR.astype(jnp.float32)
    k1 = (rank_b == iR_f).astype(jnp.float32)
    k2 = (rank_b == iR_f + 1.0).astype(jnp.float32)
    sorted_row = jnp.sum(k1 * cC, axis=0, keepdims=True)
    next_row = jnp.sum(k2 * cC, axis=0, keepdims=True)
    mid_row = 0.5 * (sorted_row + next_row)
    lane = lax.broadcasted_iota(jnp.int32, (1, C), 1)
    mid_row = jnp.where(lane == C - 1, jnp.float32(jnp.inf), mid_row)
    out_ref[...] = jnp.concatenate(
        [jnp.broadcast_to(sorted_row, (4, C)),
         jnp.broadcast_to(mid_row, (4, C))], axis=0)


_prep = pl.pallas_call(
    _prep_body,
    out_shape=jax.ShapeDtypeStruct((8, C), jnp.float32),
)


_QUANT_CACHE = []


def _get_quantize():
    if _QUANT_CACHE:
        return _QUANT_CACHE[0]

    @functools.partial(
        pl.kernel,
        mesh=plsc.VectorSubcoreMesh(core_axis_name="c", subcore_axis_name="s"),
        compiler_params=pltpu.CompilerParams(needs_layout_passes=False),
        out_type=jax.ShapeDtypeStruct((N_TOTAL,), jnp.float32),
        scratch_types=[
            pltpu.VMEM((CHUNK,), jnp.float32),
            pltpu.VMEM((C,), jnp.float32),
            pltpu.VMEM((C,), jnp.float32),
            pltpu.VMEM((CHUNK,), jnp.float32),
        ],
    )
    def _quantize(x_hbm, sorted_hbm, mid_hbm, out_hbm, xv, sv, mv, qv):
        wid = lax.axis_index("s") * NC + lax.axis_index("c")
        base = wid * CHUNK
        pltpu.sync_copy(x_hbm.at[pl.ds(base, CHUNK)], xv)
        pltpu.sync_copy(sorted_hbm, sv)
        pltpu.sync_copy(mid_hbm, mv)

        TOP = 16
        STEP0 = C // TOP
        splats = [
            plsc.load_gather(mv, [jnp.full((L,), j * STEP0 + STEP0 - 1,
                                           jnp.int32)])
            for j in range(TOP - 1)
        ]

        def search(xvec):
            pos = jnp.zeros((L,), jnp.int32)
            for sp in splats:
                pos = pos + jnp.where(sp < xvec, STEP0, 0)
            step = STEP0 // 2
            while step >= 1:
                mvals = plsc.load_gather(mv, [pos + (step - 1)])
                pos = jnp.where(mvals < xvec, pos + step, pos)
                step //= 2
            return plsc.load_gather(sv, [pos])

        UNROLL = 4
        def body(i, carry):
            b0 = i * (UNROLL * L)
            xvecs = [xv[pl.ds(b0 + u * L, L)] for u in range(UNROLL)]
            qvals = [search(xu) for xu in xvecs]
            for u in range(UNROLL):
                qv[pl.ds(b0 + u * L, L)] = qvals[u]
            return carry

        lax.fori_loop(0, NVEC // UNROLL, body, 0)
        pltpu.sync_copy(qv, out_hbm.at[pl.ds(base, CHUNK)])

    _QUANT_CACHE.append(_quantize)
    return _quantize


def _mm_body(qx_ref, w_ref, b_ref, out_ref):
    acc = lax.dot_general(
        qx_ref[...], w_ref[...], (((1,), (1,)), ((), ())),
        preferred_element_type=jnp.float32)
    out_ref[...] = acc + jnp.broadcast_to(b_ref[0:1, :], acc.shape)


def kernel(x, centers, W, b):
    ccol = jnp.broadcast_to(centers, (C, 128))
    crow = jnp.broadcast_to(centers.reshape(1, C), (8, C))
    prep = _prep(ccol, crow)
    sorted_c = prep[0]
    mid = prep[4]
    qx = _get_quantize()(x.reshape(-1), sorted_c, mid).reshape(x.shape)
    b2 = jnp.broadcast_to(b.reshape(1, -1), (8, b.shape[0]))
    out = pl.pallas_call(
        _mm_body,
        out_shape=jax.ShapeDtypeStruct((x.shape[0], W.shape[0]), jnp.float32),
    )(qx, W, b2)
    return out
